# baseline (speedup 1.0000x reference)
import functools

import jax
import jax.numpy as jnp
from jax import lax
from jax.experimental import pallas as pl
from jax.experimental.pallas import tpu as pltpu
from jax.experimental.pallas import tpu_sc as plsc

DIM = 16
LANES = 16
RPS = 8
CHUNK = 256


@functools.cache
def _build(batch, dim):
    info = plsc.get_sparse_core_info()
    nc, ns = info.num_cores, info.num_subcores
    nw = nc * ns
    assert batch % (CHUNK * nw) == 0 and dim == LANES
    bpw = batch // nw
    nchunks = bpw // CHUNK
    groups = CHUNK // LANES

    mesh = plsc.VectorSubcoreMesh(core_axis_name="c", subcore_axis_name="s")

    @functools.partial(
        pl.kernel,
        mesh=mesh,
        compiler_params=pltpu.CompilerParams(needs_layout_passes=False),
        out_type=jax.ShapeDtypeStruct((batch,), jnp.float32),
        scratch_types=[
            pltpu.VMEM((bpw,), jnp.int32),
            pltpu.VMEM((bpw,), jnp.int32),
            pltpu.VMEM((bpw,), jnp.int32),
            pltpu.VMEM((bpw,), jnp.int32),
            pltpu.VMEM((CHUNK, RPS * LANES), jnp.float32),
            pltpu.VMEM((CHUNK, RPS * LANES), jnp.float32),
            pltpu.VMEM((bpw,), jnp.float32),
            pltpu.SemaphoreType.DMA,
        ],
    )
    def mf(user_hbm, item_hbm, utab_hbm, itab_hbm, out_hbm,
           uidx_v, iidx_v, umaj_v, imaj_v, ublk_v, iblk_v, out_v, sem):
        wid = lax.axis_index("s") * nc + lax.axis_index("c")
        base = wid * bpw
        pltpu.sync_copy(user_hbm.at[pl.ds(base, bpw)], uidx_v)
        pltpu.sync_copy(item_hbm.at[pl.ds(base, bpw)], iidx_v)

        def smaj(i, _):
            umaj_v[pl.ds(i * LANES, LANES)] = (
                jnp.right_shift(uidx_v[pl.ds(i * LANES, LANES)], 3))
            imaj_v[pl.ds(i * LANES, LANES)] = (
                jnp.right_shift(iidx_v[pl.ds(i * LANES, LANES)], 3))
            return _

        lax.fori_loop(0, bpw // LANES, smaj, None)

        lane = lax.iota(jnp.int32, LANES)

        def chunk_body(c, _):
            cu = pltpu.async_copy(
                utab_hbm.at[umaj_v.at[pl.ds(c * CHUNK, CHUNK)]], ublk_v, sem)
            ci = pltpu.async_copy(
                itab_hbm.at[imaj_v.at[pl.ds(c * CHUNK, CHUNK)]], iblk_v, sem)
            cu.wait()
            ci.wait()

            def body(g, _):
                rows = g * LANES + lane
                ui = uidx_v[pl.ds(c * CHUNK + g * LANES, LANES)]
                vi = iidx_v[pl.ds(c * CHUNK + g * LANES, LANES)]
                ucol = (ui & (RPS - 1)) * LANES
                vcol = (vi & (RPS - 1)) * LANES
                acc = jnp.zeros((LANES,), jnp.float32)
                for k in range(dim):
                    u = plsc.load_gather(ublk_v, [rows, ucol + k])
                    v = plsc.load_gather(iblk_v, [rows, vcol + k])
                    acc = acc + u * v
                out_v[pl.ds(c * CHUNK + g * LANES, LANES)] = acc
                return _

            lax.fori_loop(0, groups, body, None)
            return _

        lax.fori_loop(0, nchunks, chunk_body, None)
        pltpu.sync_copy(out_v, out_hbm.at[pl.ds(base, bpw)])

    return mf


def kernel(user, item, user_table, item_table):
    batch = user.shape[0]
    dim = user_table.shape[1]
    mf = _build(batch, dim)
    utab = user_table.reshape(-1, RPS * dim)
    itab = item_table.reshape(-1, RPS * dim)
    score = mf(user.astype(jnp.int32), item.astype(jnp.int32), utab, itab)
    return score[:, None]
